# Initial kernel scaffold; baseline (speedup 1.0000x reference)
import functools
import math

import jax
import jax.numpy as jnp
from jax import lax
from jax.experimental import pallas as pl
from jax.experimental.pallas import tpu as pltpu
from jax.experimental.pallas import tpu_sc as plsc

D_MODEL = 64
MAX_REL = 30
_NUM_CORES = 2
_NUM_SUBCORES = 16
_NUM_WORKERS = _NUM_CORES * _NUM_SUBCORES


def _ext_table(q_len: int) -> jax.Array:
    positions = jnp.arange(0, 2 * MAX_REL + 1, dtype=jnp.float32)[:, None]
    div_term = jnp.exp(
        jnp.arange(0, D_MODEL, 2, dtype=jnp.float32)
        * -(math.log(10000.0) / D_MODEL)
    )
    enc = jnp.zeros((2 * MAX_REL + 1, D_MODEL), dtype=jnp.float32)
    enc = enc.at[:, 0::2].set(jnp.sin(positions * div_term))
    enc = enc.at[:, 1::2].set(jnp.cos(positions * div_term))
    t = jnp.arange(2 * q_len - 1)
    idx = jnp.clip(t - (q_len - 1), -MAX_REL, MAX_REL) + MAX_REL
    return jnp.take(enc, idx, axis=0)


def kernel(q):
    q_len = q.shape[2]
    dtype = q.dtype
    rows_per_w = q_len // _NUM_WORKERS
    win = rows_per_w - 1 + q_len

    ext = _ext_table(q_len).astype(dtype)

    mesh = plsc.VectorSubcoreMesh(core_axis_name="c", subcore_axis_name="s")

    @functools.partial(
        pl.kernel,
        out_type=jax.ShapeDtypeStruct((q_len, q_len, D_MODEL), dtype),
        mesh=mesh,
        scratch_types=[
            pltpu.VMEM((win, D_MODEL), dtype),
            pltpu.SemaphoreType.DMA,
        ],
    )
    def sc_fill(ext_hbm, out_hbm, buf, sem):
        wid = lax.axis_index("s") * _NUM_CORES + lax.axis_index("c")
        r0 = wid * rows_per_w
        lo = (q_len - 1) - r0 - (rows_per_w - 1)
        pltpu.sync_copy(ext_hbm.at[pl.ds(lo, win)], buf)
        copies = []
        for r in range(rows_per_w):
            src = buf.at[pl.ds(rows_per_w - 1 - r, q_len)]
            copies.append(pltpu.async_copy(src, out_hbm.at[r0 + r], sem))
        for c in copies:
            c.wait()

    out = sc_fill(ext)
    return out[None]


# speedup vs baseline: 5.4458x; 5.4458x over previous
import functools
import math

import jax
import jax.numpy as jnp
from jax import lax
from jax.experimental import pallas as pl
from jax.experimental.pallas import tpu as pltpu
from jax.experimental.pallas import tpu_sc as plsc

D_MODEL = 64
MAX_REL = 30
_NUM_CORES = 2
_NUM_SUBCORES = 16
_NUM_WORKERS = _NUM_CORES * _NUM_SUBCORES


def _ext_table(q_len: int) -> jax.Array:
    positions = jnp.arange(0, 2 * MAX_REL + 1, dtype=jnp.float32)[:, None]
    div_term = jnp.exp(
        jnp.arange(0, D_MODEL, 2, dtype=jnp.float32)
        * -(math.log(10000.0) / D_MODEL)
    )
    enc = jnp.zeros((2 * MAX_REL + 1, D_MODEL), dtype=jnp.float32)
    enc = enc.at[:, 0::2].set(jnp.sin(positions * div_term))
    enc = enc.at[:, 1::2].set(jnp.cos(positions * div_term))
    t = jnp.arange(2 * q_len - 1)
    idx = jnp.clip(t - (q_len - 1), -MAX_REL, MAX_REL) + MAX_REL
    return jnp.take(enc, idx, axis=0)


def kernel(q):
    q_len = q.shape[2]
    dtype = q.dtype
    rows_per_w = q_len // _NUM_WORKERS
    win = rows_per_w + q_len
    row_w = q_len * D_MODEL

    ext = _ext_table(q_len).astype(dtype)
    ext = jnp.concatenate([ext, ext[-1:]], axis=0).reshape(-1)

    mesh = plsc.VectorSubcoreMesh(core_axis_name="c", subcore_axis_name="s")

    @functools.partial(
        pl.kernel,
        out_type=jax.ShapeDtypeStruct((q_len * q_len * D_MODEL,), dtype),
        mesh=mesh,
        scratch_types=[
            pltpu.VMEM((win * D_MODEL,), dtype),
            pltpu.SemaphoreType.DMA,
        ],
    )
    def sc_fill(ext_hbm, out_hbm, buf, sem):
        wid = lax.axis_index("s") * _NUM_CORES + lax.axis_index("c")
        r0 = wid * rows_per_w
        lo = (q_len - 1) - r0 - (rows_per_w - 1)
        pltpu.sync_copy(ext_hbm.at[pl.ds(lo * D_MODEL, win * D_MODEL)], buf)
        copies = []
        for r in range(rows_per_w):
            src = buf.at[pl.ds((rows_per_w - 1 - r) * D_MODEL, row_w)]
            dst = out_hbm.at[pl.ds((r0 + r) * row_w, row_w)]
            copies.append(pltpu.async_copy(src, dst, sem))
        for c in copies:
            c.wait()

    out = sc_fill(ext)
    return out.reshape(1, q_len, q_len, D_MODEL)


# speedup vs baseline: 5.4503x; 1.0008x over previous
import functools
import math

import jax
import jax.numpy as jnp
from jax import lax
from jax.experimental import pallas as pl
from jax.experimental.pallas import tpu as pltpu
from jax.experimental.pallas import tpu_sc as plsc

D_MODEL = 64
MAX_REL = 30
_NUM_CORES = 2
_NUM_SUBCORES = 16
_NUM_WORKERS = _NUM_CORES * _NUM_SUBCORES


def _ext_table(q_len: int) -> jax.Array:
    positions = jnp.arange(0, 2 * MAX_REL + 1, dtype=jnp.float32)[:, None]
    div_term = jnp.exp(
        jnp.arange(0, D_MODEL, 2, dtype=jnp.float32)
        * -(math.log(10000.0) / D_MODEL)
    )
    enc = jnp.zeros((2 * MAX_REL + 1, D_MODEL), dtype=jnp.float32)
    enc = enc.at[:, 0::2].set(jnp.sin(positions * div_term))
    enc = enc.at[:, 1::2].set(jnp.cos(positions * div_term))
    t = jnp.arange(2 * q_len - 1)
    idx = jnp.clip(t - (q_len - 1), -MAX_REL, MAX_REL) + MAX_REL
    return jnp.take(enc, idx, axis=0)


def kernel(q):
    q_len = q.shape[2]
    dtype = q.dtype
    rows_per_w = q_len // _NUM_WORKERS
    win = rows_per_w + q_len

    ext = _ext_table(q_len).astype(dtype)
    ext = jnp.concatenate([ext, ext[-1:]], axis=0)

    mesh = plsc.VectorSubcoreMesh(core_axis_name="c", subcore_axis_name="s")

    @functools.partial(
        pl.kernel,
        out_type=jax.ShapeDtypeStruct((q_len, q_len, D_MODEL), dtype),
        mesh=mesh,
        scratch_types=[
            pltpu.VMEM((win, D_MODEL), dtype),
            pltpu.SemaphoreType.DMA,
        ],
        compiler_params=pltpu.CompilerParams(use_tc_tiling_on_sc=False),
    )
    def sc_fill(ext_hbm, out_hbm, buf, sem):
        wid = lax.axis_index("s") * _NUM_CORES + lax.axis_index("c")
        r0 = wid * rows_per_w
        lo = (q_len - 1) - r0 - (rows_per_w - 1)
        pltpu.sync_copy(ext_hbm.at[pl.ds(lo, win)], buf)
        copies = []
        for r in range(rows_per_w):
            src = buf.at[pl.ds(rows_per_w - 1 - r, q_len)]
            dst = out_hbm.at[r0 + r]
            copies.append(pltpu.async_copy(src, dst, sem))
        for c in copies:
            c.wait()

    out = sc_fill(ext)
    return out[None]


# speedup vs baseline: 10.0419x; 1.8424x over previous
import functools
import math

import jax
import jax.numpy as jnp
from jax import lax
from jax.experimental import pallas as pl
from jax.experimental.pallas import tpu as pltpu
from jax.experimental.pallas import tpu_sc as plsc

D_MODEL = 64
MAX_REL = 30
_NUM_CORES = 2
_NUM_SUBCORES = 16
_NUM_WORKERS = _NUM_CORES * _NUM_SUBCORES
_RUNBUF = 384


def _tables(q_len: int):
    positions = jnp.arange(0, 2 * MAX_REL + 1, dtype=jnp.float32)[:, None]
    div_term = jnp.exp(
        jnp.arange(0, D_MODEL, 2, dtype=jnp.float32)
        * -(math.log(10000.0) / D_MODEL)
    )
    enc = jnp.zeros((2 * MAX_REL + 1, D_MODEL), dtype=jnp.float32)
    enc = enc.at[:, 0::2].set(jnp.sin(positions * div_term))
    enc = enc.at[:, 1::2].set(jnp.cos(positions * div_term))
    t = jnp.arange(2 * q_len - 1)
    idx = jnp.clip(t - (q_len - 1), -MAX_REL, MAX_REL) + MAX_REL
    ext = jnp.take(enc, idx, axis=0)

    p = jnp.arange(128)[:, None]
    jj = jnp.arange(256)[None, :]
    m_rows = (q_len - 1) - p + jj
    m2_rows = m_rows - 128
    mlib = jnp.transpose(jnp.take(ext, m_rows, axis=0), (0, 2, 1))
    m2lib = jnp.transpose(jnp.take(ext, m2_rows, axis=0), (0, 2, 1))
    b0row = jnp.broadcast_to(enc[0][:, None], (D_MODEL, _RUNBUF))
    b60row = jnp.broadcast_to(enc[2 * MAX_REL][:, None], (D_MODEL, _RUNBUF))
    return mlib, m2lib, b0row, b60row


def kernel(q):
    q_len = q.shape[2]
    dtype = q.dtype
    mlib, m2lib, b0row, b60row = _tables(q_len)
    mlib = mlib.astype(dtype)
    m2lib = m2lib.astype(dtype)
    b0row = b0row.astype(dtype)
    b60row = b60row.astype(dtype)

    mesh = plsc.VectorSubcoreMesh(core_axis_name="c", subcore_axis_name="s")

    @functools.partial(
        pl.kernel,
        out_type=jax.ShapeDtypeStruct((q_len, D_MODEL, q_len), dtype),
        mesh=mesh,
        scratch_types=[
            pltpu.VMEM((D_MODEL, _RUNBUF), dtype),
            pltpu.VMEM((D_MODEL, _RUNBUF), dtype),
            pltpu.VMEM((2, 2, D_MODEL, 256), dtype),
            pltpu.SemaphoreType.DMA,
            pltpu.SemaphoreType.DMA,
        ],
    )
    def sc_fill(mlib_hbm, m2lib_hbm, b0_hbm, b60_hbm, out_hbm,
                b0buf, b60buf, mbuf, sem_in, sem_out):
        wid = lax.axis_index("s") * _NUM_CORES + lax.axis_index("c")
        p0 = wid * 4
        pltpu.sync_copy(b0_hbm, b0buf)
        pltpu.sync_copy(b60_hbm, b60buf)

        def emit_runs(i, buf, lo, hi, ds):
            off = lo
            while off < hi:
                w = min(_RUNBUF, hi - off)
                ds.append(pltpu.async_copy(
                    buf.at[:, pl.ds(0, w)],
                    out_hbm.at[i, :, pl.ds(off, w)], sem_out))
                off += w

        m_descs = [
            pltpu.async_copy(mlib_hbm.at[p0], mbuf.at[0, 0], sem_in),
            pltpu.async_copy(m2lib_hbm.at[p0], mbuf.at[0, 1], sem_in),
        ]
        for pi in range(4):
            p = p0 + pi
            b = pi % 2
            for d in m_descs:
                d.wait()
            ds = []
            for k in range(8):
                i = p + 128 * k
                if k > 0:
                    ds.append(pltpu.async_copy(
                        mbuf.at[b, 1],
                        out_hbm.at[i, :, pl.ds(128 * (k - 1), 256)],
                        sem_out))
                    emit_runs(i, b0buf, 0, 128 * (k - 1), ds)
                if k < 7:
                    ds.append(pltpu.async_copy(
                        mbuf.at[b, 0],
                        out_hbm.at[i, :, pl.ds(128 * k, 256)],
                        sem_out))
                    emit_runs(i, b60buf, 128 * (k + 2), q_len, ds)
            for d in ds:
                d.wait()
            if pi < 3:
                nb = (pi + 1) % 2
                m_descs = [
                    pltpu.async_copy(
                        mlib_hbm.at[p + 1], mbuf.at[nb, 0], sem_in),
                    pltpu.async_copy(
                        m2lib_hbm.at[p + 1], mbuf.at[nb, 1], sem_in),
                ]

    out = sc_fill(mlib, m2lib, b0row, b60row)
    return jnp.transpose(out, (0, 2, 1))[None]


# speedup vs baseline: 26.6171x; 2.6506x over previous
import functools
import math

import jax
import jax.numpy as jnp
from jax import lax
from jax.experimental import pallas as pl
from jax.experimental.pallas import tpu as pltpu
from jax.experimental.pallas import tpu_sc as plsc

D_MODEL = 64
MAX_REL = 30
_NUM_CORES = 2
_NUM_SUBCORES = 16
_NUM_WORKERS = _NUM_CORES * _NUM_SUBCORES
_RUNBUF = 384
_SBW = 384


def _tables(q_len: int):
    positions = jnp.arange(0, 2 * MAX_REL + 1, dtype=jnp.float32)[:, None]
    div_term = jnp.exp(
        jnp.arange(0, D_MODEL, 2, dtype=jnp.float32)
        * -(math.log(10000.0) / D_MODEL)
    )
    enc = jnp.zeros((2 * MAX_REL + 1, D_MODEL), dtype=jnp.float32)
    enc = enc.at[:, 0::2].set(jnp.sin(positions * div_term))
    enc = enc.at[:, 1::2].set(jnp.cos(positions * div_term))
    t = jnp.arange(2 * q_len - 1)
    idx = jnp.clip(t - (q_len - 1), -MAX_REL, MAX_REL) + MAX_REL
    ext_t = jnp.take(enc, idx, axis=0).T

    base = q_len - 129
    slib = jnp.stack(
        [lax.slice_in_dim(ext_t, base - p, base - p + _SBW, axis=1)
         for p in range(128)]
    )
    b0row = jnp.broadcast_to(enc[0][:, None], (D_MODEL, _RUNBUF))
    b60row = jnp.broadcast_to(enc[2 * MAX_REL][:, None], (D_MODEL, _RUNBUF))
    return slib, b0row, b60row


def kernel(q):
    q_len = q.shape[2]
    dtype = q.dtype
    slib, b0row, b60row = _tables(q_len)
    slib = slib.astype(dtype)
    b0row = b0row.astype(dtype)
    b60row = b60row.astype(dtype)

    mesh = plsc.VectorSubcoreMesh(core_axis_name="c", subcore_axis_name="s")

    @functools.partial(
        pl.kernel,
        out_type=jax.ShapeDtypeStruct((q_len, D_MODEL, q_len), dtype),
        mesh=mesh,
        scratch_types=[
            pltpu.VMEM((D_MODEL, _RUNBUF), dtype),
            pltpu.VMEM((D_MODEL, _RUNBUF), dtype),
            pltpu.VMEM((2, D_MODEL, _SBW), dtype),
            pltpu.SemaphoreType.DMA,
            pltpu.SemaphoreType.DMA,
        ],
    )
    def sc_fill(slib_hbm, b0_hbm, b60_hbm, out_hbm,
                b0buf, b60buf, mbuf, sem_in, sem_out):
        wid = lax.axis_index("s") * _NUM_CORES + lax.axis_index("c")
        p0 = wid * 4
        pltpu.sync_copy(b0_hbm, b0buf)
        pltpu.sync_copy(b60_hbm, b60buf)

        def emit_runs(i, buf, lo, hi, ds):
            off = lo
            while off < hi:
                w = min(_RUNBUF, hi - off)
                ds.append(pltpu.async_copy(
                    buf.at[:, pl.ds(0, w)],
                    out_hbm.at[i, :, pl.ds(off, w)], sem_out))
                off += w

        stage = pltpu.async_copy(slib_hbm.at[p0], mbuf.at[0], sem_in)
        bg = []
        prev_mixed = []
        for pi in range(4):
            p = p0 + pi
            b = pi % 2
            stage.wait()
            mixed = []
            for k in range(8):
                i = p + 128 * k
                if k == 0:
                    src, off, w = mbuf.at[b, :, pl.ds(128, 256)], 0, 256
                elif k == 7:
                    src, off, w = mbuf.at[b, :, pl.ds(0, 256)], 768, 256
                else:
                    src, off, w = mbuf.at[b], 128 * (k - 1), _SBW
                mixed.append(pltpu.async_copy(
                    src, out_hbm.at[i, :, pl.ds(off, w)], sem_out))
                emit_runs(i, b0buf, 0, off, bg)
                emit_runs(i, b60buf, off + w, q_len, bg)
            if pi < 3:
                for d in prev_mixed:
                    d.wait()
                stage = pltpu.async_copy(
                    slib_hbm.at[p + 1], mbuf.at[(pi + 1) % 2], sem_in)
            prev_mixed, mixed = mixed, None
        for d in prev_mixed:
            d.wait()
        for d in bg:
            d.wait()

    out = sc_fill(slib, b0row, b60row)
    return jnp.transpose(out, (0, 2, 1))[None]


# speedup vs baseline: 27.0153x; 1.0150x over previous
import functools
import math

import jax
import jax.numpy as jnp
from jax import lax
from jax.experimental import pallas as pl
from jax.experimental.pallas import tpu as pltpu
from jax.experimental.pallas import tpu_sc as plsc

D_MODEL = 64
MAX_REL = 30
_NUM_CORES = 2
_NUM_SUBCORES = 16
_NUM_WORKERS = _NUM_CORES * _NUM_SUBCORES
_RUNBUF = 384
_SBW = 384


def _tables(q_len: int):
    positions = jnp.arange(0, 2 * MAX_REL + 1, dtype=jnp.float32)[:, None]
    div_term = jnp.exp(
        jnp.arange(0, D_MODEL, 2, dtype=jnp.float32)
        * -(math.log(10000.0) / D_MODEL)
    )
    enc = jnp.zeros((2 * MAX_REL + 1, D_MODEL), dtype=jnp.float32)
    enc = enc.at[:, 0::2].set(jnp.sin(positions * div_term))
    enc = enc.at[:, 1::2].set(jnp.cos(positions * div_term))
    t = jnp.arange(2 * q_len - 1)
    idx = jnp.clip(t - (q_len - 1), -MAX_REL, MAX_REL) + MAX_REL
    ext_t = jnp.take(enc, idx, axis=0).T

    base = q_len - 129
    slib = jnp.stack(
        [lax.slice_in_dim(ext_t, base - p, base - p + _SBW, axis=1)
         for p in range(128)]
    )
    b0row = jnp.broadcast_to(enc[0][:, None], (D_MODEL, _RUNBUF))
    b60row = jnp.broadcast_to(enc[2 * MAX_REL][:, None], (D_MODEL, _RUNBUF))
    return slib, b0row, b60row


def kernel(q):
    q_len = q.shape[2]
    dtype = q.dtype
    slib, b0row, b60row = _tables(q_len)
    slib = slib.astype(dtype)
    b0row = b0row.astype(dtype)
    b60row = b60row.astype(dtype)

    mesh = plsc.VectorSubcoreMesh(core_axis_name="c", subcore_axis_name="s")

    @functools.partial(
        pl.kernel,
        out_type=jax.ShapeDtypeStruct((q_len, D_MODEL, q_len), dtype),
        mesh=mesh,
        scratch_types=[
            pltpu.VMEM((D_MODEL, _RUNBUF), dtype),
            pltpu.VMEM((D_MODEL, _RUNBUF), dtype),
            pltpu.VMEM((3, D_MODEL, _SBW), dtype),
            pltpu.SemaphoreType.DMA,
            pltpu.SemaphoreType.DMA,
        ],
    )
    def sc_fill(slib_hbm, b0_hbm, b60_hbm, out_hbm,
                b0buf, b60buf, mbuf, sem_in, sem_out):
        wid = lax.axis_index("s") * _NUM_CORES + lax.axis_index("c")
        p0 = wid * 4
        pltpu.sync_copy(b0_hbm, b0buf)
        pltpu.sync_copy(b60_hbm, b60buf)

        def emit_runs(i, buf, lo, hi, ds):
            off = lo
            while off < hi:
                w = min(_RUNBUF, hi - off)
                ds.append(pltpu.async_copy(
                    buf.at[:, pl.ds(0, w)],
                    out_hbm.at[i, :, pl.ds(off, w)], sem_out))
                off += w

        stages = {0: pltpu.async_copy(slib_hbm.at[p0], mbuf.at[0], sem_in)}
        bg = []
        mixed = {}
        for pi in range(4):
            p = p0 + pi
            b = pi % 3
            stages[pi].wait()
            if pi < 3:
                if pi >= 2:
                    for d in mixed[pi - 2]:
                        d.wait()
                stages[pi + 1] = pltpu.async_copy(
                    slib_hbm.at[p + 1], mbuf.at[(pi + 1) % 3], sem_in)
            mixed[pi] = []
            for k in range(8):
                i = p + 128 * k
                if k == 0:
                    src, off, w = mbuf.at[b, :, pl.ds(128, 256)], 0, 256
                elif k == 7:
                    src, off, w = mbuf.at[b, :, pl.ds(0, 256)], 768, 256
                else:
                    src, off, w = mbuf.at[b], 128 * (k - 1), _SBW
                mixed[pi].append(pltpu.async_copy(
                    src, out_hbm.at[i, :, pl.ds(off, w)], sem_out))
                emit_runs(i, b0buf, 0, off, bg)
                emit_runs(i, b60buf, off + w, q_len, bg)
        for pi in (1, 2, 3):
            for d in mixed[pi]:
                d.wait()
        for d in bg:
            d.wait()

    out = sc_fill(slib, b0row, b60row)
    return jnp.transpose(out, (0, 2, 1))[None]
